# baseline (speedup 1.0000x reference)
import functools

import jax
import jax.numpy as jnp
from jax import lax
from jax.experimental import pallas as pl
from jax.experimental.pallas import tpu as pltpu
from jax.experimental.pallas import tpu_sc as plsc

N = 10000
E = 320000
IN = 128
OUT = 64
H1 = 42
H2 = 21

NC = 2
NS = 16
NW = NC * NS
L = 16

NP = 10240
EPW = E // NW
C = 80
SUB = 4
CC = SUB * C
NIT = EPW // CC
NROW = EPW // C
GPI = CC // L
GPS = C // L
W = OUT + L
ROWS_PT = NP // NS


def _sc_compute_groups(ngroups, xl_rows, xr_rows, sc_buf, att_v, tr_v, ex_v,
                       lane):
    att_regs = [att_v[pl.ds(L + k * L, L)] for k in range(OUT // L)]

    @pl.loop(0, ngroups)
    def _(g):
        row0 = g * L
        for e in range(L):
            r = row0 + e
            acc = None
            for k in range(OUT // L):
                a = xl_rows[r, pl.ds(k * L, L)]
                b = xr_rows[r, pl.ds(k * L, L)]
                z = a + b
                lz = jnp.maximum(z, 0.2 * z)
                t = att_regs[k] * lz
                acc = t if acc is None else acc + t
            plsc.store_scatter(tr_v, [lane, jnp.full((L,), e, jnp.int32)],
                               acc)
        tot = None
        for d in range(L):
            row = tr_v[d, pl.ds(0, L)]
            tot = row if tot is None else tot + row
        ex = jnp.exp(tot)
        plsc.store_scatter(sc_buf, [lane + row0, jnp.full((L,), OUT, jnp.int32)],
                           ex)
        ex_v[pl.ds(L, L)] = ex
        for e in range(L):
            r = row0 + e
            exe = plsc.load_gather(ex_v, [jnp.full((L,), L + e, jnp.int32)])
            for k in range(OUT // L):
                sc_buf[r, pl.ds(k * L, L)] = exe * xl_rows[r, pl.ds(k * L, L)]


def _sc_edge_body(xl_hbm, xr_hbm, att_hbm, src_hbm, dst_hbm,
                  oun_out,
                  src2, dst2, xl_rows, xr_rows, sc_buf, att_v, tr_v, ex_v,
                  out_sh, semi, semg, semsc):
    c = lax.axis_index("c")
    s = lax.axis_index("s")
    wid = s * NC + c

    zero16 = jnp.zeros((L,), jnp.float32)

    @pl.loop(0, CC * W // L)
    def _(j):
        r = j // (W // L)
        k = j % (W // L)
        sc_buf[r, pl.ds(k * L, L)] = zero16

    for j in range(ROWS_PT // CC):
        pltpu.sync_copy(sc_buf,
                        out_sh.at[pl.ds(s * ROWS_PT + j * CC, CC)])

    pltpu.sync_copy(att_hbm, att_v.at[pl.ds(L, OUT)])
    plsc.subcore_barrier()

    lane = lax.iota(jnp.int32, L)

    def _idx_pair(i, parity):
        base = jnp.minimum(i * SUB, NROW - 1 - SUB)
        a = pltpu.make_async_copy(src_hbm.at[wid, pl.ds(base, SUB)],
                                  src2.at[pl.ds(parity * SUB, SUB)], semi)
        b = pltpu.make_async_copy(dst_hbm.at[wid, pl.ds(base, SUB)],
                                  dst2.at[pl.ds(parity * SUB, SUB)], semi)
        return a, b

    def fire_idx(i, parity):
        a, b = _idx_pair(i, parity)
        a.start()
        b.start()

    def drain_idx(i, parity):
        a, b = _idx_pair(i, parity)
        a.wait()
        b.wait()

    fire_idx(jnp.int32(0), jnp.int32(0))

    @pl.loop(0, NIT)
    def _(i):
        p = lax.rem(i, 2)
        drain_idx(i, p)
        fire_idx(i + 1, 1 - p)
        descs = []
        for k in range(SUB):
            descs.append(pltpu.async_copy(
                xl_hbm.at[src2.at[p * SUB + k]],
                xl_rows.at[pl.ds(k * C, C)], semg))
            descs.append(pltpu.async_copy(
                xr_hbm.at[dst2.at[p * SUB + k]],
                xr_rows.at[pl.ds(k * C, C)], semg))
        for dsc in descs:
            dsc.wait()

        _sc_compute_groups(GPI, xl_rows, xr_rows, sc_buf, att_v, tr_v, ex_v,
                           lane)

        sdescs = []
        for k in range(SUB):
            sdescs.append(pltpu.async_copy(
                sc_buf.at[pl.ds(k * C, C)],
                out_sh.at[dst2.at[p * SUB + k]], semsc, add=True))
        for dsc in sdescs:
            dsc.wait()

    drain_idx(jnp.int32(0), jnp.int32(NIT % 2))

    pltpu.sync_copy(src_hbm.at[wid, pl.ds(NIT * SUB, 1)],
                    src2.at[pl.ds(0, 1)])
    pltpu.sync_copy(dst_hbm.at[wid, pl.ds(NIT * SUB, 1)],
                    dst2.at[pl.ds(0, 1)])
    pltpu.async_copy(xl_hbm.at[src2.at[0]],
                     xl_rows.at[pl.ds(0, C)], semg).wait()
    pltpu.async_copy(xr_hbm.at[dst2.at[0]],
                     xr_rows.at[pl.ds(0, C)], semg).wait()
    _sc_compute_groups(GPS, xl_rows, xr_rows, sc_buf, att_v, tr_v, ex_v,
                       lane)
    pltpu.async_copy(sc_buf.at[pl.ds(0, C)],
                     out_sh.at[dst2.at[0]], semsc, add=True).wait()

    plsc.subcore_barrier()
    col0 = s * ROWS_PT
    pltpu.sync_copy(out_sh.at[pl.ds(col0, ROWS_PT)],
                    oun_out.at[c, pl.ds(col0, ROWS_PT)])


def _sc_edge(xl, xr, att, src, dst):
    mesh = plsc.VectorSubcoreMesh(core_axis_name="c", subcore_axis_name="s")
    f = pl.kernel(
        _sc_edge_body,
        out_type=jax.ShapeDtypeStruct((NC, NP, W), jnp.float32),
        mesh=mesh,
        compiler_params=pltpu.CompilerParams(
            needs_layout_passes=False, use_tc_tiling_on_sc=False),
        scratch_types=[
            pltpu.VMEM((2 * SUB, C), jnp.int32),
            pltpu.VMEM((2 * SUB, C), jnp.int32),
            pltpu.VMEM((CC, OUT), jnp.float32),
            pltpu.VMEM((CC, OUT), jnp.float32),
            pltpu.VMEM((CC, W), jnp.float32),
            pltpu.VMEM((OUT + L,), jnp.float32),
            pltpu.VMEM((L, L), jnp.float32),
            pltpu.VMEM((2 * L,), jnp.float32),
            pltpu.VMEM_SHARED((NP, W), jnp.float32),
            pltpu.SemaphoreType.DMA,
            pltpu.SemaphoreType.DMA,
            pltpu.SemaphoreType.DMA,
        ],
    )
    return f(xl, xr, att, src.reshape(NW, NROW, C), dst.reshape(NW, NROW, C))


def _bn_cols(x, g, b, eps=1e-5):
    m = jnp.mean(x, axis=0, keepdims=True)
    v = jnp.mean((x - m) ** 2, axis=0, keepdims=True)
    return g * (x - m) * jax.lax.rsqrt(v + eps) + b


def _tc1_body(x_ref, gin_ref, bin_ref, w_ref, b_ref, y_ref, q_ref):
    xn = _bn_cols(x_ref[...], gin_ref[...], bin_ref[...])
    y = jnp.dot(xn, w_ref[...], preferred_element_type=jnp.float32) + b_ref[...]
    y_ref[...] = y[:, : 6 * OUT]
    q_ref[...] = jnp.tanh(y[:, 6 * OUT:])


def _tc1(x, gin, bin_, w_all, b_all):
    return pl.pallas_call(
        _tc1_body,
        out_shape=(
            jax.ShapeDtypeStruct((N, 6 * OUT), jnp.float32),
            jax.ShapeDtypeStruct((N, OUT), jnp.float32),
        ),
    )(x, gin.reshape(1, IN), bin_.reshape(1, IN), w_all, b_all.reshape(1, -1))


def _tc2a_body(o_ref, bias_ref, g_ref, b_ref, q_ref,
               wk_ref, bk_ref, wv_ref, bv_ref, s_ref, vals_ref):
    acc = o_ref[0, :N, :] + o_ref[1, :N, :]
    den = acc[:, OUT] + 1e-16
    oun = acc[:, :OUT]
    o = oun / den[:, None] + bias_ref[...]
    e = jnp.tanh(_bn_cols(o, g_ref[...], b_ref[...]))
    keys = jnp.tanh(
        jnp.dot(e, wk_ref[...], preferred_element_type=jnp.float32)
        + bk_ref[...])
    s_ref[...] = jnp.sum(keys * q_ref[...], axis=1, keepdims=True)
    vals_ref[...] = jnp.tanh(
        jnp.dot(e, wv_ref[...], preferred_element_type=jnp.float32)
        + bv_ref[...])


def _tc2a(oun, bias, g, b, q, p):
    return pl.pallas_call(
        _tc2a_body,
        out_shape=(
            jax.ShapeDtypeStruct((N, 1), jnp.float32),
            jax.ShapeDtypeStruct((N, OUT), jnp.float32),
        ),
    )(oun, bias.reshape(1, OUT), g.reshape(1, OUT), b.reshape(1, OUT),
      q, p['Wk'], p['bk'].reshape(1, OUT), p['Wv'], p['bv'].reshape(1, OUT))


def _tc2b_body(sp_ref, vp_ref, ss_ref, vs_ref, sv_ref, vv_ref,
               wc1_ref, bc1_ref, g1_ref, bn1_ref,
               wc2_ref, bc2_ref, g2_ref, bn2_ref,
               wc3_ref, bc3_ref, out_ref):
    ws = [sp_ref[...], ss_ref[...], sv_ref[...]]
    vals = [vp_ref[...], vs_ref[...], vv_ref[...]]
    m = jnp.maximum(jnp.maximum(ws[0], ws[1]), ws[2])
    es = [jnp.exp(w - m) for w in ws]
    tot = es[0] + es[1] + es[2]
    r = (es[0] * vals[0] + es[1] * vals[1] + es[2] * vals[2]) / tot
    h = jnp.tanh(_bn_cols(
        jnp.dot(r, wc1_ref[...], preferred_element_type=jnp.float32)
        + bc1_ref[...], g1_ref[...], bn1_ref[...]))
    h = jnp.tanh(_bn_cols(
        jnp.dot(h, wc2_ref[...], preferred_element_type=jnp.float32)
        + bc2_ref[...], g2_ref[...], bn2_ref[...]))
    out_ref[...] = (
        jnp.dot(h, wc3_ref[...], preferred_element_type=jnp.float32)
        + bc3_ref[...])


def _tc2b(svs, p):
    args = []
    for (s, v) in svs:
        args += [s, v]
    return pl.pallas_call(
        _tc2b_body,
        out_shape=jax.ShapeDtypeStruct((N, 2), jnp.float32),
    )(*args,
      p['Wc1'], p['bc1'].reshape(1, H1), p['gc1'].reshape(1, H1),
      p['bnc1'].reshape(1, H1),
      p['Wc2'], p['bc2'].reshape(1, H2), p['gc2'].reshape(1, H2),
      p['bnc2'].reshape(1, H2),
      p['Wc3'], p['bc3'].reshape(1, 2))


def kernel(x, edge_index_p, edge_index_s, edge_index_v, params):
    p = params
    w_all = jnp.concatenate(
        [p['p_Wl'], p['p_Wr'], p['s_Wl'], p['s_Wr'],
         p['v_Wl'], p['v_Wr'], p['Wq']], axis=1)
    b_all = jnp.concatenate(
        [p['p_bl'], p['p_br'], p['s_bl'], p['s_br'],
         p['v_bl'], p['v_br'], p['bq']], axis=0)
    y, q = _tc1(x, p['g_in'], p['b_in'], w_all, b_all)

    svs = []
    for i, (pref, ei) in enumerate(
            (('p', edge_index_p), ('s', edge_index_s), ('v', edge_index_v))):
        xl = y[:, 2 * i * OUT:(2 * i + 1) * OUT]
        xr = y[:, (2 * i + 1) * OUT:(2 * i + 2) * OUT]
        oun = _sc_edge(xl, xr, p[pref + '_att'], ei[0], ei[1])
        svs.append(_tc2a(oun, p[pref + '_bias'], p[pref + '_g'],
                         p[pref + '_b'], q, p))

    return _tc2b(svs, params)


# speedup vs baseline: 2.1933x; 2.1933x over previous
import functools

import jax
import jax.numpy as jnp
from jax import lax
from jax.experimental import pallas as pl
from jax.experimental.pallas import tpu as pltpu
from jax.experimental.pallas import tpu_sc as plsc

N = 10000
E = 320000
IN = 128
OUT = 64
H1 = 42
H2 = 21

NC = 2
NS = 16
NW = NC * NS
L = 16

NP = 10240
EPW = E // NW
C = 80
NROW = EPW // C
GPS = C // L
W = OUT + L
ROWS_PT = NP // NS


def _sc_edge_body(xl_hbm, xr_hbm, att_hbm, src_hbm, dst_hbm,
                  oun_out,
                  src_all, dst_all, xl_rows, xr_rows, sc_buf, att_v,
                  out_sh, semg0, semg1, semsc0, semsc1):
    c = lax.axis_index("c")
    s = lax.axis_index("s")
    wid = s * NC + c

    zero16 = jnp.zeros((L,), jnp.float32)

    @pl.loop(0, 2 * C * W // L)
    def _(j):
        r = j // (W // L)
        k = j % (W // L)
        sc_buf[r, pl.ds(k * L, L)] = zero16

    for j in range(ROWS_PT // (2 * C)):
        pltpu.sync_copy(sc_buf,
                        out_sh.at[pl.ds(s * ROWS_PT + j * 2 * C, 2 * C)])

    pltpu.sync_copy(att_hbm, att_v.at[pl.ds(L, OUT)])
    pltpu.sync_copy(src_hbm.at[wid], src_all)
    pltpu.sync_copy(dst_hbm.at[wid], dst_all)
    plsc.subcore_barrier()

    lane = lax.iota(jnp.int32, L)
    att_regs = [att_v[pl.ds(L + k * L, L)] for k in range(OUT // L)]

    def compute_half(base):
        for g in range(GPS):
            for e in range(L):
                r = base + g * L + e
                xls = []
                acc = None
                for k in range(OUT // L):
                    a = xl_rows[r, pl.ds(k * L, L)]
                    b = xr_rows[r, pl.ds(k * L, L)]
                    xls.append(a)
                    z = a + b
                    lz = jnp.maximum(z, 0.2 * z)
                    t = att_regs[k] * lz
                    acc = t if acc is None else acc + t
                sv = jnp.sum(acc)
                ex = jnp.exp(jnp.broadcast_to(sv, (L,)))
                for k in range(OUT // L):
                    sc_buf[r, pl.ds(k * L, L)] = ex * xls[k]
                sc_buf[r, pl.ds(OUT, L)] = jnp.where(lane == 0, ex, 0.0)

    sems_g = (semg0, semg1)
    sems_sc = (semsc0, semsc1)

    def gather_pair(h, half):
        off = half * C
        a = pltpu.make_async_copy(xl_hbm.at[src_all.at[h]],
                                  xl_rows.at[pl.ds(off, C)], sems_g[half])
        b = pltpu.make_async_copy(xr_hbm.at[dst_all.at[h]],
                                  xr_rows.at[pl.ds(off, C)], sems_g[half])
        return a, b

    def fire_gather(h, half):
        a, b = gather_pair(h, half)
        a.start()
        b.start()

    def drain_gather(h, half):
        a, b = gather_pair(h, half)
        a.wait()
        b.wait()

    def scatter_desc(h, half):
        off = half * C
        return pltpu.make_async_copy(sc_buf.at[pl.ds(off, C)],
                                     out_sh.at[dst_all.at[h]], sems_sc[half])

    fire_gather(jnp.int32(0), 0)
    fire_gather(jnp.int32(1), 1)

    @pl.loop(0, (NROW - 1) // 2)
    def _(j):
        h0 = 2 * j
        for half in range(2):
            h = h0 + half

            @pl.when(j > 0)
            def _():
                scatter_desc(h, half).wait()

            drain_gather(h, half)
            compute_half(half * C)
            scatter_desc(h, half).start(add=True)
            fire_gather(jnp.minimum(h + 2, NROW - 1), half)

    last = jnp.int32(NROW - 1)
    scatter_desc(last, 0).wait()
    scatter_desc(last, 1).wait()
    drain_gather(last, 0)
    drain_gather(last, 1)
    compute_half(0)
    sd = scatter_desc(last, 0)
    sd.start(add=True)
    sd.wait()

    plsc.subcore_barrier()
    col0 = s * ROWS_PT
    pltpu.sync_copy(out_sh.at[pl.ds(col0, ROWS_PT)],
                    oun_out.at[c, pl.ds(col0, ROWS_PT)])


def _sc_edge(xl, xr, att, src, dst):
    mesh = plsc.VectorSubcoreMesh(core_axis_name="c", subcore_axis_name="s")
    f = pl.kernel(
        _sc_edge_body,
        out_type=jax.ShapeDtypeStruct((NC, NP, W), jnp.float32),
        mesh=mesh,
        compiler_params=pltpu.CompilerParams(
            needs_layout_passes=False, use_tc_tiling_on_sc=False),
        scratch_types=[
            pltpu.VMEM((NROW, C), jnp.int32),
            pltpu.VMEM((NROW, C), jnp.int32),
            pltpu.VMEM((2 * C, OUT), jnp.float32),
            pltpu.VMEM((2 * C, OUT), jnp.float32),
            pltpu.VMEM((2 * C, W), jnp.float32),
            pltpu.VMEM((OUT + L,), jnp.float32),
            pltpu.VMEM_SHARED((NP, W), jnp.float32),
            pltpu.SemaphoreType.DMA,
            pltpu.SemaphoreType.DMA,
            pltpu.SemaphoreType.DMA,
            pltpu.SemaphoreType.DMA,
        ],
    )
    return f(xl, xr, att, src.reshape(NW, NROW, C), dst.reshape(NW, NROW, C))


def _bn_cols(x, g, b, eps=1e-5):
    m = jnp.mean(x, axis=0, keepdims=True)
    v = jnp.mean((x - m) ** 2, axis=0, keepdims=True)
    return g * (x - m) * jax.lax.rsqrt(v + eps) + b


def _tc1_body(x_ref, gin_ref, bin_ref, w_ref, b_ref, y_ref, q_ref):
    xn = _bn_cols(x_ref[...], gin_ref[...], bin_ref[...])
    y = jnp.dot(xn, w_ref[...], preferred_element_type=jnp.float32) + b_ref[...]
    y_ref[...] = y[:, : 6 * OUT]
    q_ref[...] = jnp.tanh(y[:, 6 * OUT:])


def _tc1(x, gin, bin_, w_all, b_all):
    return pl.pallas_call(
        _tc1_body,
        out_shape=(
            jax.ShapeDtypeStruct((N, 6 * OUT), jnp.float32),
            jax.ShapeDtypeStruct((N, OUT), jnp.float32),
        ),
    )(x, gin.reshape(1, IN), bin_.reshape(1, IN), w_all, b_all.reshape(1, -1))


def _tc2a_body(o_ref, bias_ref, g_ref, b_ref, q_ref,
               wk_ref, bk_ref, wv_ref, bv_ref, s_ref, vals_ref):
    acc = o_ref[0, :N, :] + o_ref[1, :N, :]
    den = acc[:, OUT] + 1e-16
    oun = acc[:, :OUT]
    o = oun / den[:, None] + bias_ref[...]
    e = jnp.tanh(_bn_cols(o, g_ref[...], b_ref[...]))
    keys = jnp.tanh(
        jnp.dot(e, wk_ref[...], preferred_element_type=jnp.float32)
        + bk_ref[...])
    s_ref[...] = jnp.sum(keys * q_ref[...], axis=1, keepdims=True)
    vals_ref[...] = jnp.tanh(
        jnp.dot(e, wv_ref[...], preferred_element_type=jnp.float32)
        + bv_ref[...])


def _tc2a(oun, bias, g, b, q, p):
    return pl.pallas_call(
        _tc2a_body,
        out_shape=(
            jax.ShapeDtypeStruct((N, 1), jnp.float32),
            jax.ShapeDtypeStruct((N, OUT), jnp.float32),
        ),
    )(oun, bias.reshape(1, OUT), g.reshape(1, OUT), b.reshape(1, OUT),
      q, p['Wk'], p['bk'].reshape(1, OUT), p['Wv'], p['bv'].reshape(1, OUT))


def _tc2b_body(sp_ref, vp_ref, ss_ref, vs_ref, sv_ref, vv_ref,
               wc1_ref, bc1_ref, g1_ref, bn1_ref,
               wc2_ref, bc2_ref, g2_ref, bn2_ref,
               wc3_ref, bc3_ref, out_ref):
    ws = [sp_ref[...], ss_ref[...], sv_ref[...]]
    vals = [vp_ref[...], vs_ref[...], vv_ref[...]]
    m = jnp.maximum(jnp.maximum(ws[0], ws[1]), ws[2])
    es = [jnp.exp(w - m) for w in ws]
    tot = es[0] + es[1] + es[2]
    r = (es[0] * vals[0] + es[1] * vals[1] + es[2] * vals[2]) / tot
    h = jnp.tanh(_bn_cols(
        jnp.dot(r, wc1_ref[...], preferred_element_type=jnp.float32)
        + bc1_ref[...], g1_ref[...], bn1_ref[...]))
    h = jnp.tanh(_bn_cols(
        jnp.dot(h, wc2_ref[...], preferred_element_type=jnp.float32)
        + bc2_ref[...], g2_ref[...], bn2_ref[...]))
    out_ref[...] = (
        jnp.dot(h, wc3_ref[...], preferred_element_type=jnp.float32)
        + bc3_ref[...])


def _tc2b(svs, p):
    args = []
    for (s, v) in svs:
        args += [s, v]
    return pl.pallas_call(
        _tc2b_body,
        out_shape=jax.ShapeDtypeStruct((N, 2), jnp.float32),
    )(*args,
      p['Wc1'], p['bc1'].reshape(1, H1), p['gc1'].reshape(1, H1),
      p['bnc1'].reshape(1, H1),
      p['Wc2'], p['bc2'].reshape(1, H2), p['gc2'].reshape(1, H2),
      p['bnc2'].reshape(1, H2),
      p['Wc3'], p['bc3'].reshape(1, 2))


def kernel(x, edge_index_p, edge_index_s, edge_index_v, params):
    p = params
    w_all = jnp.concatenate(
        [p['p_Wl'], p['p_Wr'], p['s_Wl'], p['s_Wr'],
         p['v_Wl'], p['v_Wr'], p['Wq']], axis=1)
    b_all = jnp.concatenate(
        [p['p_bl'], p['p_br'], p['s_bl'], p['s_br'],
         p['v_bl'], p['v_br'], p['bq']], axis=0)
    y, q = _tc1(x, p['g_in'], p['b_in'], w_all, b_all)

    svs = []
    for i, (pref, ei) in enumerate(
            (('p', edge_index_p), ('s', edge_index_s), ('v', edge_index_v))):
        xl = y[:, 2 * i * OUT:(2 * i + 1) * OUT]
        xr = y[:, (2 * i + 1) * OUT:(2 * i + 2) * OUT]
        oun = _sc_edge(xl, xr, p[pref + '_att'], ei[0], ei[1])
        svs.append(_tc2a(oun, p[pref + '_bias'], p[pref + '_g'],
                         p[pref + '_b'], q, p))

    return _tc2b(svs, params)
